# baseline (speedup 1.0000x reference)
import functools

import jax
import jax.numpy as jnp
from jax import lax
from jax.experimental import pallas as pl
from jax.experimental.pallas import tpu as pltpu
from jax.experimental.pallas import tpu_sc as plsc

D_MODEL = 1024
D_FF = 4096
E = 8
K = 2
T = 4096
BT = 512
NB = T // BT * 2 + E
NSLOT = NB * BT
BF = 1024
NF = D_FF // BF
NW = 32
TPW = T // NW
CH = 32
NCH = TPW // CH


def _router_kernel(x_ref, gw_ref, slot1_ref, slot2_ref, wn1_ref,
                   wn2_ref, be_ref, nbu_ref, aux_ref):
    x = x_ref[...]
    gw = gw_ref[...]
    logits = lax.dot_general(
        x, gw, (((1,), (1,)), ((), ())), preferred_element_type=jnp.float32
    )
    l1 = jnp.max(logits, axis=-1, keepdims=True)
    ex = jnp.exp(logits - l1)
    probs = ex / jnp.sum(ex, axis=-1, keepdims=True)

    iota = lax.broadcasted_iota(jnp.int32, (T, E), 1)
    i1 = jnp.min(jnp.where(logits == l1, iota, E), axis=-1, keepdims=True)
    masked = jnp.where(iota == i1, -jnp.inf, logits)
    l2 = jnp.max(masked, axis=-1, keepdims=True)
    i2 = jnp.min(jnp.where(masked == l2, iota, E), axis=-1, keepdims=True)

    p1 = jnp.sum(jnp.where(iota == i1, probs, 0.0), axis=-1, keepdims=True)
    p2 = jnp.sum(jnp.where(iota == i2, probs, 0.0), axis=-1, keepdims=True)
    s = p1 + p2
    wn1_ref[...] = p1 / s
    wn2_ref[...] = p2 / s

    oh1 = (iota == i1)
    oh2 = (iota == i2)
    oh = (oh1 | oh2).astype(jnp.float32)

    inc = oh
    sh = 1
    while sh < T:
        inc = inc + jnp.concatenate(
            [jnp.zeros((sh, E), jnp.float32), inc[: T - sh]], axis=0
        )
        sh *= 2
    r1 = jnp.sum(jnp.where(oh1, inc, 0.0), axis=-1, keepdims=True) - 1.0
    r2 = jnp.sum(jnp.where(oh2, inc, 0.0), axis=-1, keepdims=True) - 1.0

    counts = jnp.sum(oh, axis=0, keepdims=True)
    ci = counts.astype(jnp.int32)
    bc = (ci + (BT - 1)) // BT
    bend = bc
    for shl in (1, 2, 4):
        bend = bend + jnp.concatenate(
            [jnp.zeros((1, shl), jnp.int32), bend[:, : E - shl]], axis=1
        )
    off = (bend - bc) * BT

    offb = jnp.broadcast_to(off, (T, E))
    s1 = jnp.sum(jnp.where(oh1, offb, 0), axis=-1, keepdims=True)
    s2 = jnp.sum(jnp.where(oh2, offb, 0), axis=-1, keepdims=True)
    slot1_ref[...] = s1 + r1.astype(jnp.int32)
    slot2_ref[...] = s2 + r2.astype(jnp.int32)

    iota_nb = lax.broadcasted_iota(jnp.int32, (NB, E), 0)
    bendb = jnp.broadcast_to(bend, (NB, E))
    be = jnp.sum((iota_nb >= bendb).astype(jnp.int32), axis=-1, keepdims=True)
    be_ref[...] = jnp.minimum(be, E - 1)
    nbu_ref[0, 0] = jnp.sum(bc)

    importance = jnp.mean(probs, axis=0, keepdims=True)
    aux_ref[0, 0] = E * jnp.sum(importance * counts) / (T * K)


def _sc_scatter_kernel(x_hbm, slots_hbm, xs_hbm, idx_v, rows_a, rows_b,
                       sem_in, sem_out):
    wid = lax.axis_index("s") * 2 + lax.axis_index("c")
    pltpu.sync_copy(slots_hbm.at[wid], idx_v)
    bufs = (rows_a, rows_b)
    base = wid * TPW
    loads = [None] * NCH
    scats = [None] * NCH
    loads[0] = pltpu.async_copy(x_hbm.at[pl.ds(base, CH)], bufs[0], sem_in)
    for j in range(NCH):
        loads[j].wait()
        b = bufs[j % 2]
        scats[j] = (
            pltpu.async_copy(b, xs_hbm.at[idx_v.at[j]], sem_out),
            pltpu.async_copy(b, xs_hbm.at[idx_v.at[NCH + j]], sem_out),
        )
        if j + 1 < NCH:
            if j >= 1:
                scats[j - 1][0].wait()
                scats[j - 1][1].wait()
            loads[j + 1] = pltpu.async_copy(
                x_hbm.at[pl.ds(base + (j + 1) * CH, CH)], bufs[(j + 1) % 2],
                sem_in,
            )
    for c in (NCH - 2, NCH - 1):
        scats[c][0].wait()
        scats[c][1].wait()


def _sc_gather_kernel(ys_hbm, slots_hbm, g1_hbm, g2_hbm, idx_v, rows_a,
                      rows_b, sem):
    wid = lax.axis_index("s") * 2 + lax.axis_index("c")
    pltpu.sync_copy(slots_hbm.at[wid], idx_v)
    bufs = (rows_a, rows_b)
    gets = [None] * (2 * NCH)
    gets[0] = pltpu.async_copy(ys_hbm.at[idx_v.at[0]], bufs[0], sem)
    for c in range(2 * NCH):
        if c + 1 < 2 * NCH:
            gets[c + 1] = pltpu.async_copy(
                ys_hbm.at[idx_v.at[c + 1]], bufs[(c + 1) % 2], sem
            )
        gets[c].wait()
        dst = g1_hbm if c < NCH else g2_hbm
        j = c % NCH
        pltpu.sync_copy(bufs[c % 2], dst.at[pl.ds(wid * TPW + j * CH, CH)])


def _h_kernel(meta_ref, xs_ref, w1_ref, b1_ref, h_ref, w1s_ref):
    f = pl.program_id(0)
    i = pl.program_id(1)

    @pl.when(i < meta_ref[NB])
    def _():
        prev = meta_ref[jnp.maximum(i - 1, 0)]

        @pl.when(jnp.logical_or(i == 0, meta_ref[i] != prev))
        def _cast():
            w1s_ref[...] = w1_ref[0].astype(jnp.bfloat16)

        xb = xs_ref[...].astype(jnp.bfloat16)
        h = lax.dot_general(
            xb, w1s_ref[...], (((1,), (1,)), ((), ())),
            preferred_element_type=jnp.float32,
        )
        h = h + b1_ref[0]
        inner = 0.7978845608028654 * (h + 0.044715 * h * h * h)
        h_ref[...] = (0.5 * h * (1.0 + jnp.tanh(inner))).astype(jnp.bfloat16)


def _y_kernel(meta_ref, h_ref, w2_ref, b2_ref, ys_ref, w2s_ref):
    i = pl.program_id(0)

    @pl.when(i < meta_ref[NB])
    def _():
        prev = meta_ref[jnp.maximum(i - 1, 0)]

        @pl.when(jnp.logical_or(i == 0, meta_ref[i] != prev))
        def _cast():
            w2s_ref[...] = w2_ref[0].astype(jnp.bfloat16)

        hb = h_ref[...]
        y = lax.dot_general(
            hb, w2s_ref[...], (((1,), (1,)), ((), ())),
            preferred_element_type=jnp.float32,
        )
        ys_ref[...] = y + b2_ref[0]


def _combine_kernel(g1_ref, g2_ref, wn1_ref, wn2_ref, out_ref):
    out_ref[...] = g1_ref[...] * wn1_ref[...] + g2_ref[...] * wn2_ref[...]


@functools.partial(jax.jit, static_argnames=())
def kernel(hidden_states, gate_w, w1, b1, w2, b2):
    orig_shape = hidden_states.shape
    x = hidden_states.reshape(T, D_MODEL)

    slot1, slot2, wn1, wn2, be, nbu, aux = pl.pallas_call(
        _router_kernel,
        out_shape=(
            jax.ShapeDtypeStruct((T, 1), jnp.int32),
            jax.ShapeDtypeStruct((T, 1), jnp.int32),
            jax.ShapeDtypeStruct((T, 1), jnp.float32),
            jax.ShapeDtypeStruct((T, 1), jnp.float32),
            jax.ShapeDtypeStruct((NB, 1), jnp.int32),
            jax.ShapeDtypeStruct((1, 1), jnp.int32),
            jax.ShapeDtypeStruct((1, 1), jnp.float32),
        ),
        out_specs=(
            pl.BlockSpec(memory_space=pltpu.VMEM),
            pl.BlockSpec(memory_space=pltpu.VMEM),
            pl.BlockSpec(memory_space=pltpu.VMEM),
            pl.BlockSpec(memory_space=pltpu.VMEM),
            pl.BlockSpec(memory_space=pltpu.VMEM),
            pl.BlockSpec(memory_space=pltpu.SMEM),
            pl.BlockSpec(memory_space=pltpu.SMEM),
        ),
    )(x, gate_w)

    slots3d = jnp.concatenate(
        [slot1.reshape(NW, NCH, CH), slot2.reshape(NW, NCH, CH)], axis=1
    )
    meta = jnp.concatenate([be.reshape(NB), nbu.reshape(1)])

    mesh = plsc.VectorSubcoreMesh(core_axis_name="c", subcore_axis_name="s")

    sc_scatter = functools.partial(
        pl.kernel,
        mesh=mesh,
        out_type=jax.ShapeDtypeStruct((NSLOT, D_MODEL), jnp.float32),
        scratch_types=[
            pltpu.VMEM((2 * NCH, CH), jnp.int32),
            pltpu.VMEM((CH, D_MODEL), jnp.float32),
            pltpu.VMEM((CH, D_MODEL), jnp.float32),
            pltpu.SemaphoreType.DMA,
            pltpu.SemaphoreType.DMA,
        ],
    )(_sc_scatter_kernel)
    xs = sc_scatter(x, slots3d)

    hmat = pl.pallas_call(
        _h_kernel,
        grid_spec=pltpu.PrefetchScalarGridSpec(
            num_scalar_prefetch=1,
            grid=(NF, NB),
            in_specs=[
                pl.BlockSpec((BT, D_MODEL), lambda f, i, m: (i, 0)),
                pl.BlockSpec((1, BF, D_MODEL), lambda f, i, m: (m[i], f, 0)),
                pl.BlockSpec((1, 1, BF), lambda f, i, m: (m[i], 0, f)),
            ],
            out_specs=pl.BlockSpec((BT, BF), lambda f, i, m: (i, f)),
            scratch_shapes=[
                pltpu.VMEM((BF, D_MODEL), jnp.bfloat16),
            ],
        ),
        out_shape=jax.ShapeDtypeStruct((NSLOT, D_FF), jnp.bfloat16),
    )(meta, xs, w1, b1.reshape(E, 1, D_FF))

    ys = pl.pallas_call(
        _y_kernel,
        grid_spec=pltpu.PrefetchScalarGridSpec(
            num_scalar_prefetch=1,
            grid=(NB,),
            in_specs=[
                pl.BlockSpec((BT, D_FF), lambda i, m: (i, 0)),
                pl.BlockSpec((1, D_MODEL, D_FF), lambda i, m: (m[i], 0, 0)),
                pl.BlockSpec((1, 1, D_MODEL), lambda i, m: (m[i], 0, 0)),
            ],
            out_specs=pl.BlockSpec((BT, D_MODEL), lambda i, m: (i, 0)),
            scratch_shapes=[
                pltpu.VMEM((D_MODEL, D_FF), jnp.bfloat16),
            ],
        ),
        out_shape=jax.ShapeDtypeStruct((NSLOT, D_MODEL), jnp.float32),
    )(meta, hmat, w2, b2.reshape(E, 1, D_MODEL))

    sc_gather = functools.partial(
        pl.kernel,
        mesh=mesh,
        out_type=(
            jax.ShapeDtypeStruct((T, D_MODEL), jnp.float32),
            jax.ShapeDtypeStruct((T, D_MODEL), jnp.float32),
        ),
        scratch_types=[
            pltpu.VMEM((2 * NCH, CH), jnp.int32),
            pltpu.VMEM((CH, D_MODEL), jnp.float32),
            pltpu.VMEM((CH, D_MODEL), jnp.float32),
            pltpu.SemaphoreType.DMA,
        ],
    )(_sc_gather_kernel)
    g1, g2 = sc_gather(ys, slots3d)

    BTC = 1024
    out = pl.pallas_call(
        _combine_kernel,
        grid=(T // BTC,),
        in_specs=[
            pl.BlockSpec((BTC, D_MODEL), lambda i: (i, 0)),
            pl.BlockSpec((BTC, D_MODEL), lambda i: (i, 0)),
            pl.BlockSpec((BTC, 1), lambda i: (i, 0)),
            pl.BlockSpec((BTC, 1), lambda i: (i, 0)),
        ],
        out_specs=pl.BlockSpec((BTC, D_MODEL), lambda i: (i, 0)),
        out_shape=jax.ShapeDtypeStruct((T, D_MODEL), jnp.float32),
    )(g1, g2, wn1, wn2)

    return out.reshape(orig_shape), aux.reshape(())


# speedup vs baseline: 1.1237x; 1.1237x over previous
import functools

import jax
import jax.numpy as jnp
from jax import lax
from jax.experimental import pallas as pl
from jax.experimental.pallas import tpu as pltpu
from jax.experimental.pallas import tpu_sc as plsc

D_MODEL = 1024
D_FF = 4096
E = 8
K = 2
T = 4096
BT = 512
NB = T // BT * 2 + E
NSLOT = NB * BT
BF = 1024
NF = D_FF // BF
NW = 32
TPW = T // NW
CH = 32
NCH = TPW // CH


def _router_kernel(x_ref, gw_ref, slot1_ref, slot2_ref, wn1_ref,
                   wn2_ref, be_ref, nbu_ref, aux_ref):
    x = x_ref[...]
    gw = gw_ref[...]
    logits = lax.dot_general(
        x, gw, (((1,), (1,)), ((), ())), preferred_element_type=jnp.float32
    )
    l1 = jnp.max(logits, axis=-1, keepdims=True)
    ex = jnp.exp(logits - l1)
    probs = ex / jnp.sum(ex, axis=-1, keepdims=True)

    iota = lax.broadcasted_iota(jnp.int32, (T, E), 1)
    i1 = jnp.min(jnp.where(logits == l1, iota, E), axis=-1, keepdims=True)
    masked = jnp.where(iota == i1, -jnp.inf, logits)
    l2 = jnp.max(masked, axis=-1, keepdims=True)
    i2 = jnp.min(jnp.where(masked == l2, iota, E), axis=-1, keepdims=True)

    p1 = jnp.sum(jnp.where(iota == i1, probs, 0.0), axis=-1, keepdims=True)
    p2 = jnp.sum(jnp.where(iota == i2, probs, 0.0), axis=-1, keepdims=True)
    s = p1 + p2
    wn1_ref[...] = p1 / s
    wn2_ref[...] = p2 / s

    oh1 = (iota == i1)
    oh2 = (iota == i2)
    oh = (oh1 | oh2).astype(jnp.float32)

    inc = oh
    sh = 1
    while sh < T:
        inc = inc + jnp.concatenate(
            [jnp.zeros((sh, E), jnp.float32), inc[: T - sh]], axis=0
        )
        sh *= 2
    r1 = jnp.sum(jnp.where(oh1, inc, 0.0), axis=-1, keepdims=True) - 1.0
    r2 = jnp.sum(jnp.where(oh2, inc, 0.0), axis=-1, keepdims=True) - 1.0

    counts = jnp.sum(oh, axis=0, keepdims=True)
    ci = counts.astype(jnp.int32)
    bc = (ci + (BT - 1)) // BT
    bend = bc
    for shl in (1, 2, 4):
        bend = bend + jnp.concatenate(
            [jnp.zeros((1, shl), jnp.int32), bend[:, : E - shl]], axis=1
        )
    off = (bend - bc) * BT

    offb = jnp.broadcast_to(off, (T, E))
    s1 = jnp.sum(jnp.where(oh1, offb, 0), axis=-1, keepdims=True)
    s2 = jnp.sum(jnp.where(oh2, offb, 0), axis=-1, keepdims=True)
    slot1_ref[...] = s1 + r1.astype(jnp.int32)
    slot2_ref[...] = s2 + r2.astype(jnp.int32)

    iota_nb = lax.broadcasted_iota(jnp.int32, (NB, E), 0)
    bendb = jnp.broadcast_to(bend, (NB, E))
    be = jnp.sum((iota_nb >= bendb).astype(jnp.int32), axis=-1, keepdims=True)
    be_ref[...] = jnp.minimum(be, E - 1)
    nbu_ref[0, 0] = jnp.sum(bc)

    importance = jnp.mean(probs, axis=0, keepdims=True)
    aux_ref[0, 0] = E * jnp.sum(importance * counts) / (T * K)


def _sc_scatter_kernel(x_hbm, slots_hbm, xs_hbm, idx_v, rows_a, rows_b,
                       sem_in, sem_out):
    wid = lax.axis_index("s") * 2 + lax.axis_index("c")
    pltpu.sync_copy(slots_hbm.at[wid], idx_v)
    bufs = (rows_a, rows_b)
    base = wid * TPW
    loads = [None] * NCH
    scats = [None] * NCH
    loads[0] = pltpu.async_copy(x_hbm.at[pl.ds(base, CH)], bufs[0], sem_in)
    for j in range(NCH):
        loads[j].wait()
        b = bufs[j % 2]
        scats[j] = (
            pltpu.async_copy(b, xs_hbm.at[idx_v.at[j]], sem_out),
            pltpu.async_copy(b, xs_hbm.at[idx_v.at[NCH + j]], sem_out),
        )
        if j + 1 < NCH:
            if j >= 1:
                scats[j - 1][0].wait()
                scats[j - 1][1].wait()
            loads[j + 1] = pltpu.async_copy(
                x_hbm.at[pl.ds(base + (j + 1) * CH, CH)], bufs[(j + 1) % 2],
                sem_in,
            )
    for c in (NCH - 2, NCH - 1):
        scats[c][0].wait()
        scats[c][1].wait()


def _sc_gather_kernel(ys_hbm, slots_hbm, g1_hbm, g2_hbm, idx_v, rows_a,
                      rows_b, sem):
    wid = lax.axis_index("s") * 2 + lax.axis_index("c")
    pltpu.sync_copy(slots_hbm.at[wid], idx_v)
    bufs = (rows_a, rows_b)
    gets = [None] * (2 * NCH)
    gets[0] = pltpu.async_copy(ys_hbm.at[idx_v.at[0]], bufs[0], sem)
    for c in range(2 * NCH):
        if c + 1 < 2 * NCH:
            gets[c + 1] = pltpu.async_copy(
                ys_hbm.at[idx_v.at[c + 1]], bufs[(c + 1) % 2], sem
            )
        gets[c].wait()
        dst = g1_hbm if c < NCH else g2_hbm
        j = c % NCH
        pltpu.sync_copy(bufs[c % 2], dst.at[pl.ds(wid * TPW + j * CH, CH)])


def _ffn_kernel(meta_ref, xs_ref, w1_ref, b1_ref, w2_ref, b2_ref, ys_ref):
    i = pl.program_id(0)

    @pl.when(i < meta_ref[NB])
    def _():
        xb = xs_ref[...].astype(jnp.bfloat16)
        h = lax.dot_general(
            xb, w1_ref[0], (((1,), (1,)), ((), ())),
            preferred_element_type=jnp.float32,
        )
        h = h + b1_ref[0]
        inner = 0.7978845608028654 * (h + 0.044715 * h * h * h)
        h = (0.5 * h * (1.0 + jnp.tanh(inner))).astype(jnp.bfloat16)
        y = lax.dot_general(
            h, w2_ref[0], (((1,), (1,)), ((), ())),
            preferred_element_type=jnp.float32,
        )
        ys_ref[...] = y + b2_ref[0]


def _combine_kernel(g1_ref, g2_ref, wn1_ref, wn2_ref, out_ref):
    out_ref[...] = g1_ref[...] * wn1_ref[...] + g2_ref[...] * wn2_ref[...]


@functools.partial(jax.jit, static_argnames=())
def kernel(hidden_states, gate_w, w1, b1, w2, b2):
    orig_shape = hidden_states.shape
    x = hidden_states.reshape(T, D_MODEL)

    slot1, slot2, wn1, wn2, be, nbu, aux = pl.pallas_call(
        _router_kernel,
        out_shape=(
            jax.ShapeDtypeStruct((T, 1), jnp.int32),
            jax.ShapeDtypeStruct((T, 1), jnp.int32),
            jax.ShapeDtypeStruct((T, 1), jnp.float32),
            jax.ShapeDtypeStruct((T, 1), jnp.float32),
            jax.ShapeDtypeStruct((NB, 1), jnp.int32),
            jax.ShapeDtypeStruct((1, 1), jnp.int32),
            jax.ShapeDtypeStruct((1, 1), jnp.float32),
        ),
        out_specs=(
            pl.BlockSpec(memory_space=pltpu.VMEM),
            pl.BlockSpec(memory_space=pltpu.VMEM),
            pl.BlockSpec(memory_space=pltpu.VMEM),
            pl.BlockSpec(memory_space=pltpu.VMEM),
            pl.BlockSpec(memory_space=pltpu.VMEM),
            pl.BlockSpec(memory_space=pltpu.SMEM),
            pl.BlockSpec(memory_space=pltpu.SMEM),
        ),
    )(x, gate_w)

    slots3d = jnp.concatenate(
        [slot1.reshape(NW, NCH, CH), slot2.reshape(NW, NCH, CH)], axis=1
    )
    meta = jnp.concatenate([be.reshape(NB), nbu.reshape(1)])

    mesh = plsc.VectorSubcoreMesh(core_axis_name="c", subcore_axis_name="s")

    sc_scatter = functools.partial(
        pl.kernel,
        mesh=mesh,
        out_type=jax.ShapeDtypeStruct((NSLOT, D_MODEL), jnp.float32),
        scratch_types=[
            pltpu.VMEM((2 * NCH, CH), jnp.int32),
            pltpu.VMEM((CH, D_MODEL), jnp.float32),
            pltpu.VMEM((CH, D_MODEL), jnp.float32),
            pltpu.SemaphoreType.DMA,
            pltpu.SemaphoreType.DMA,
        ],
    )(_sc_scatter_kernel)
    xs = sc_scatter(x, slots3d)

    w1b = w1.astype(jnp.bfloat16)
    w2b = w2.astype(jnp.bfloat16)
    ys = pl.pallas_call(
        _ffn_kernel,
        grid_spec=pltpu.PrefetchScalarGridSpec(
            num_scalar_prefetch=1,
            grid=(NB,),
            in_specs=[
                pl.BlockSpec((BT, D_MODEL), lambda i, m: (i, 0)),
                pl.BlockSpec((1, D_FF, D_MODEL), lambda i, m: (m[i], 0, 0)),
                pl.BlockSpec((1, 1, D_FF), lambda i, m: (m[i], 0, 0)),
                pl.BlockSpec((1, D_MODEL, D_FF), lambda i, m: (m[i], 0, 0)),
                pl.BlockSpec((1, 1, D_MODEL), lambda i, m: (m[i], 0, 0)),
            ],
            out_specs=pl.BlockSpec((BT, D_MODEL), lambda i, m: (i, 0)),
        ),
        out_shape=jax.ShapeDtypeStruct((NSLOT, D_MODEL), jnp.float32),
    )(meta, xs, w1b, b1.reshape(E, 1, D_FF), w2b, b2.reshape(E, 1, D_MODEL))

    sc_gather = functools.partial(
        pl.kernel,
        mesh=mesh,
        out_type=(
            jax.ShapeDtypeStruct((T, D_MODEL), jnp.float32),
            jax.ShapeDtypeStruct((T, D_MODEL), jnp.float32),
        ),
        scratch_types=[
            pltpu.VMEM((2 * NCH, CH), jnp.int32),
            pltpu.VMEM((CH, D_MODEL), jnp.float32),
            pltpu.VMEM((CH, D_MODEL), jnp.float32),
            pltpu.SemaphoreType.DMA,
        ],
    )(_sc_gather_kernel)
    g1, g2 = sc_gather(ys, slots3d)

    BTC = 1024
    out = pl.pallas_call(
        _combine_kernel,
        grid=(T // BTC,),
        in_specs=[
            pl.BlockSpec((BTC, D_MODEL), lambda i: (i, 0)),
            pl.BlockSpec((BTC, D_MODEL), lambda i: (i, 0)),
            pl.BlockSpec((BTC, 1), lambda i: (i, 0)),
            pl.BlockSpec((BTC, 1), lambda i: (i, 0)),
        ],
        out_specs=pl.BlockSpec((BTC, D_MODEL), lambda i: (i, 0)),
        out_shape=jax.ShapeDtypeStruct((T, D_MODEL), jnp.float32),
    )(g1, g2, wn1, wn2)

    return out.reshape(orig_shape), aux.reshape(())
